# baseline (speedup 1.0000x reference)
import functools

import jax
import jax.numpy as jnp
from jax import lax
from jax.experimental import pallas as pl
from jax.experimental.pallas import tpu as pltpu
from jax.experimental.pallas import tpu_sc as plsc

BATCH = 4096
SEQ = 200
EMB = 64
VOCAB = 1000000
LANES = 16
NCOL = EMB // LANES

NC = 2
NS = 16
NW = NC * NS

VBLK = 128
NFULL = VOCAB // VBLK
TAIL = VOCAB - NFULL * VBLK
BLK_PER_W = (NFULL + NW - 1) // NW

B_PER_W = BATCH // NW
CHUNKS_PER_B = 2
CHUNK = SEQ // CHUNKS_PER_B
ROWS_PER_W = B_PER_W * CHUNKS_PER_B

_NEG = float(jnp.finfo(jnp.float32).min)
_UNROLL = 20


_TROWS = VBLK // 2
_TUNROLL = 8


@functools.partial(
    pl.kernel,
    out_type=jax.ShapeDtypeStruct((VOCAB // 2, 2 * EMB), jnp.float32),
    mesh=plsc.VectorSubcoreMesh(core_axis_name="c", subcore_axis_name="s"),
    compiler_params=pltpu.CompilerParams(needs_layout_passes=False),
    scratch_types=[
        pltpu.VMEM((EMB, VBLK + 1), jnp.float32),
        pltpu.VMEM((EMB, VBLK + 1), jnp.float32),
        pltpu.VMEM((_TROWS, VBLK), jnp.float32),
        pltpu.VMEM((_TROWS, VBLK), jnp.float32),
        pltpu.SemaphoreType.DMA,
        pltpu.SemaphoreType.DMA,
        pltpu.SemaphoreType.DMA,
        pltpu.SemaphoreType.DMA,
    ],
)
def _transpose_table(tab_t_hbm, tail_hbm, out_hbm,
                     src0, src1, dst0, dst1, in0, in1, out0, out1):
    wid = lax.axis_index("s") * NC + lax.axis_index("c")
    iota = lax.iota(jnp.int32, LANES)
    srcs, dsts = (src0, src1), (dst0, dst1)
    in_sems, out_sems = (in0, in1), (out0, out1)
    e_vecs = [LANES * jj + iota for jj in range(NCOL)]

    def start_in(c, p):
        pltpu.async_copy(
            tab_t_hbm.at[:, pl.ds(c * VBLK, VBLK)],
            srcs[p].at[:, pl.ds(0, VBLK)],
            in_sems[p],
        )

    def transpose_block(src, dst):
        def step(it, carry):
            u0v, u1v = carry
            for rr in range(_TUNROLL):
                r = it * _TUNROLL + rr
                for j in range(2 * NCOL):
                    uv = u0v if j < NCOL else u1v
                    vals = plsc.load_gather(src, [e_vecs[j % NCOL], uv])
                    dst[r, pl.ds(LANES * j, LANES)] = vals
                u0v = u0v + 2
                u1v = u1v + 2
            return u0v, u1v

        lax.fori_loop(
            0,
            _TROWS // _TUNROLL,
            step,
            (jnp.zeros((LANES,), jnp.int32), jnp.ones((LANES,), jnp.int32)),
        )

    start_in(wid, 0)
    start_in(NW + wid, 1)

    def body(m, carry):
        for p in range(2):
            c = (2 * m + p) * NW + wid

            @pl.when(c < NFULL)
            def _():
                pltpu.make_async_copy(
                    tab_t_hbm.at[:, pl.ds(c * VBLK, VBLK)],
                    srcs[p].at[:, pl.ds(0, VBLK)],
                    in_sems[p],
                ).wait()

                @pl.when(m >= 1)
                def _():
                    pltpu.make_async_copy(
                        dsts[p], out_hbm.at[pl.ds(0, _TROWS), :], out_sems[p]
                    ).wait()

                transpose_block(srcs[p], dsts[p])
                pltpu.async_copy(
                    dsts[p],
                    out_hbm.at[pl.ds(c * _TROWS, _TROWS), :],
                    out_sems[p],
                )
                cn = c + 2 * NW

                @pl.when(cn < NFULL)
                def _():
                    start_in(cn, p)

        return carry

    lax.fori_loop(0, (BLK_PER_W + 1) // 2, body, 0)

    for p in range(2):
        @pl.when(p * NW + wid < NFULL)
        def _():
            pltpu.make_async_copy(
                dsts[p], out_hbm.at[pl.ds(0, _TROWS), :], out_sems[p]
            ).wait()

    @pl.when(wid == 0)
    def _():
        pltpu.sync_copy(tail_hbm, dst0.at[pl.ds(0, TAIL // 2), :])
        pltpu.sync_copy(
            dst0.at[pl.ds(0, TAIL // 2), :],
            out_hbm.at[pl.ds(NFULL * (VBLK // 2), TAIL // 2), :],
        )


def _reduce_chunk(buf):
    def body(it, accs):
        s0 = it * _UNROLL
        for u in range(_UNROLL):
            accs = tuple(
                jnp.maximum(a, buf[s0 + u, pl.ds(LANES * j, LANES)])
                for j, a in enumerate(accs)
            )
        return accs

    init = tuple(jnp.full((LANES,), _NEG, jnp.float32) for _ in range(NCOL))
    return lax.fori_loop(0, CHUNK // _UNROLL, body, init)


@functools.partial(
    pl.kernel,
    out_type=jax.ShapeDtypeStruct((BATCH, EMB), jnp.float32),
    mesh=plsc.VectorSubcoreMesh(core_axis_name="c", subcore_axis_name="s"),
    compiler_params=pltpu.CompilerParams(use_tc_tiling_on_sc=False),
    scratch_types=[
        pltpu.VMEM((ROWS_PER_W, CHUNK), jnp.int32),
        pltpu.VMEM((CHUNK, EMB), jnp.float32),
        pltpu.VMEM((CHUNK, EMB), jnp.float32),
        pltpu.VMEM((B_PER_W, EMB), jnp.float32),
        pltpu.SemaphoreType.DMA,
        pltpu.SemaphoreType.DMA,
    ],
)
def _bow_encode(idx_hbm, table_hbm, out_hbm, idx_v, buf0, buf1, out_v, sem0, sem1):
    wid = lax.axis_index("s") * NC + lax.axis_index("c")
    base = wid * ROWS_PER_W

    pltpu.sync_copy(idx_hbm.at[pl.ds(base, ROWS_PER_W), :], idx_v)

    pltpu.async_copy(table_hbm.at[idx_v.at[0]], buf0, sem0)
    pltpu.async_copy(table_hbm.at[idx_v.at[1]], buf1, sem1)

    def gbody(g, carry):
        r0 = 2 * g

        pltpu.make_async_copy(table_hbm.at[idx_v.at[r0]], buf0, sem0).wait()
        acc0 = _reduce_chunk(buf0)

        @pl.when(g < B_PER_W - 1)
        def _():
            pltpu.async_copy(table_hbm.at[idx_v.at[r0 + 2]], buf0, sem0)

        pltpu.make_async_copy(table_hbm.at[idx_v.at[r0 + 1]], buf1, sem1).wait()
        acc1 = _reduce_chunk(buf1)

        @pl.when(g < B_PER_W - 1)
        def _():
            pltpu.async_copy(table_hbm.at[idx_v.at[r0 + 3]], buf1, sem1)

        for j in range(NCOL):
            out_v[g, pl.ds(LANES * j, LANES)] = jnp.maximum(acc0[j], acc1[j])
        return carry

    lax.fori_loop(0, B_PER_W, gbody, 0)

    pltpu.sync_copy(out_v, out_hbm.at[pl.ds(wid * B_PER_W, B_PER_W), :])


@jax.jit
def kernel(input, emb_weight):
    idx = input.astype(jnp.int32).reshape(BATCH * CHUNKS_PER_B, CHUNK)
    tail = emb_weight[NFULL * VBLK :, :].reshape(TAIL // 2, 2 * EMB)
    ctable = _transpose_table(emb_weight.T, tail)
    return _bow_encode(idx, ctable.reshape(VOCAB, EMB))


# speedup vs baseline: 1.3084x; 1.3084x over previous
import functools

import jax
import jax.numpy as jnp
from jax import lax
from jax.experimental import pallas as pl
from jax.experimental.pallas import tpu as pltpu
from jax.experimental.pallas import tpu_sc as plsc

BATCH = 4096
SEQ = 200
EMB = 64
VOCAB = 1000000
LANES = 16
NCOL = EMB // LANES

NC = 2
NS = 16
NW = NC * NS

VBLK = 128
NFULL = VOCAB // VBLK
TAIL = VOCAB - NFULL * VBLK
BLK_PER_W = (NFULL + NW - 1) // NW

B_PER_W = BATCH // NW
CHUNKS_PER_B = 2
CHUNK = SEQ // CHUNKS_PER_B
ROWS_PER_W = B_PER_W * CHUNKS_PER_B

_NEG = float(jnp.finfo(jnp.float32).min)
_UNROLL = 20


_TROWS = VBLK // 2
_TUNROLL = 4


@functools.partial(
    pl.kernel,
    out_type=jax.ShapeDtypeStruct((VOCAB // 2, 2 * EMB), jnp.float32),
    mesh=plsc.VectorSubcoreMesh(core_axis_name="c", subcore_axis_name="s"),
    compiler_params=pltpu.CompilerParams(needs_layout_passes=False),
    scratch_types=[
        pltpu.VMEM((EMB, VBLK + 1), jnp.float32),
        pltpu.VMEM((EMB, VBLK + 1), jnp.float32),
        pltpu.VMEM((_TROWS, VBLK), jnp.float32),
        pltpu.VMEM((_TROWS, VBLK), jnp.float32),
        pltpu.SemaphoreType.DMA,
        pltpu.SemaphoreType.DMA,
        pltpu.SemaphoreType.DMA,
        pltpu.SemaphoreType.DMA,
    ],
)
def _transpose_table(tab_t_hbm, tail_hbm, out_hbm,
                     src0, src1, dst0, dst1, in0, in1, out0, out1):
    wid = lax.axis_index("s") * NC + lax.axis_index("c")
    iota = lax.iota(jnp.int32, LANES)
    srcs, dsts = (src0, src1), (dst0, dst1)
    in_sems, out_sems = (in0, in1), (out0, out1)
    e_vecs = [LANES * jj + iota for jj in range(NCOL)]

    def start_in(c, p):
        pltpu.async_copy(
            tab_t_hbm.at[:, pl.ds(c * VBLK, VBLK)],
            srcs[p].at[:, pl.ds(0, VBLK)],
            in_sems[p],
        )

    def transpose_block(src, dst):
        def step(it, carry):
            u0v, u1v = carry
            rows = []
            for rr in range(_TUNROLL):
                vals = [
                    plsc.load_gather(
                        src, [e_vecs[j % NCOL], u0v if j < NCOL else u1v]
                    )
                    for j in range(2 * NCOL)
                ]
                rows.append(vals)
                u0v = u0v + 2
                u1v = u1v + 2
            for rr in range(_TUNROLL):
                r = it * _TUNROLL + rr
                for j in range(2 * NCOL):
                    dst[r, pl.ds(LANES * j, LANES)] = rows[rr][j]
            return u0v, u1v

        lax.fori_loop(
            0,
            _TROWS // _TUNROLL,
            step,
            (jnp.zeros((LANES,), jnp.int32), jnp.ones((LANES,), jnp.int32)),
        )

    start_in(wid, 0)
    start_in(NW + wid, 1)

    def body(m, carry):
        for p in range(2):
            c = (2 * m + p) * NW + wid

            @pl.when(c < NFULL)
            def _():
                pltpu.make_async_copy(
                    tab_t_hbm.at[:, pl.ds(c * VBLK, VBLK)],
                    srcs[p].at[:, pl.ds(0, VBLK)],
                    in_sems[p],
                ).wait()

                @pl.when(m >= 1)
                def _():
                    pltpu.make_async_copy(
                        dsts[p], out_hbm.at[pl.ds(0, _TROWS), :], out_sems[p]
                    ).wait()

                transpose_block(srcs[p], dsts[p])
                pltpu.async_copy(
                    dsts[p],
                    out_hbm.at[pl.ds(c * _TROWS, _TROWS), :],
                    out_sems[p],
                )
                cn = c + 2 * NW

                @pl.when(cn < NFULL)
                def _():
                    start_in(cn, p)

        return carry

    lax.fori_loop(0, (BLK_PER_W + 1) // 2, body, 0)

    for p in range(2):
        @pl.when(p * NW + wid < NFULL)
        def _():
            pltpu.make_async_copy(
                dsts[p], out_hbm.at[pl.ds(0, _TROWS), :], out_sems[p]
            ).wait()

    @pl.when(wid == 0)
    def _():
        pltpu.sync_copy(tail_hbm, dst0.at[pl.ds(0, TAIL // 2), :])
        pltpu.sync_copy(
            dst0.at[pl.ds(0, TAIL // 2), :],
            out_hbm.at[pl.ds(NFULL * (VBLK // 2), TAIL // 2), :],
        )


def _reduce_chunk(buf):
    def body(it, accs):
        s0 = it * _UNROLL
        for u in range(_UNROLL):
            accs = tuple(
                jnp.maximum(a, buf[s0 + u, pl.ds(LANES * j, LANES)])
                for j, a in enumerate(accs)
            )
        return accs

    init = tuple(jnp.full((LANES,), _NEG, jnp.float32) for _ in range(NCOL))
    return lax.fori_loop(0, CHUNK // _UNROLL, body, init)


@functools.partial(
    pl.kernel,
    out_type=jax.ShapeDtypeStruct((BATCH, EMB), jnp.float32),
    mesh=plsc.VectorSubcoreMesh(core_axis_name="c", subcore_axis_name="s"),
    compiler_params=pltpu.CompilerParams(use_tc_tiling_on_sc=False),
    scratch_types=[
        pltpu.VMEM((ROWS_PER_W, CHUNK), jnp.int32),
        pltpu.VMEM((CHUNK, EMB), jnp.float32),
        pltpu.VMEM((CHUNK, EMB), jnp.float32),
        pltpu.VMEM((B_PER_W, EMB), jnp.float32),
        pltpu.SemaphoreType.DMA,
        pltpu.SemaphoreType.DMA,
    ],
)
def _bow_encode(idx_hbm, table_hbm, out_hbm, idx_v, buf0, buf1, out_v, sem0, sem1):
    wid = lax.axis_index("s") * NC + lax.axis_index("c")
    base = wid * ROWS_PER_W

    pltpu.sync_copy(idx_hbm.at[pl.ds(base, ROWS_PER_W), :], idx_v)

    pltpu.async_copy(table_hbm.at[idx_v.at[0]], buf0, sem0)
    pltpu.async_copy(table_hbm.at[idx_v.at[1]], buf1, sem1)

    def gbody(g, carry):
        r0 = 2 * g

        pltpu.make_async_copy(table_hbm.at[idx_v.at[r0]], buf0, sem0).wait()
        acc0 = _reduce_chunk(buf0)

        @pl.when(g < B_PER_W - 1)
        def _():
            pltpu.async_copy(table_hbm.at[idx_v.at[r0 + 2]], buf0, sem0)

        pltpu.make_async_copy(table_hbm.at[idx_v.at[r0 + 1]], buf1, sem1).wait()
        acc1 = _reduce_chunk(buf1)

        @pl.when(g < B_PER_W - 1)
        def _():
            pltpu.async_copy(table_hbm.at[idx_v.at[r0 + 3]], buf1, sem1)

        for j in range(NCOL):
            out_v[g, pl.ds(LANES * j, LANES)] = jnp.maximum(acc0[j], acc1[j])
        return carry

    lax.fori_loop(0, B_PER_W, gbody, 0)

    pltpu.sync_copy(out_v, out_hbm.at[pl.ds(wid * B_PER_W, B_PER_W), :])


@jax.jit
def kernel(input, emb_weight):
    idx = input.astype(jnp.int32).reshape(BATCH * CHUNKS_PER_B, CHUNK)
    tail = emb_weight[NFULL * VBLK :, :].reshape(TAIL // 2, 2 * EMB)
    ctable = _transpose_table(emb_weight.T, tail)
    return _bow_encode(idx, ctable.reshape(VOCAB, EMB))


# speedup vs baseline: 1.4364x; 1.0979x over previous
import functools

import jax
import jax.numpy as jnp
from jax import lax
from jax.experimental import pallas as pl
from jax.experimental.pallas import tpu as pltpu
from jax.experimental.pallas import tpu_sc as plsc

BATCH = 4096
SEQ = 200
EMB = 64
VOCAB = 1000000
LANES = 16
NCOL = EMB // LANES

NC = 2
NS = 16
NW = NC * NS

VBLK = 128
NFULL = VOCAB // VBLK
TAIL = VOCAB - NFULL * VBLK
BLK_PER_W = (NFULL + NW - 1) // NW

B_PER_W = BATCH // NW
CHUNKS_PER_B = 2
CHUNK = SEQ // CHUNKS_PER_B
ROWS_PER_W = B_PER_W * CHUNKS_PER_B

_NEG = float(jnp.finfo(jnp.float32).min)
_UNROLL = 20


_TROWS = VBLK // 2
_TUNROLL = 4


@functools.partial(
    pl.kernel,
    out_type=jax.ShapeDtypeStruct((VOCAB // 2, 2 * EMB), jnp.float32),
    mesh=plsc.VectorSubcoreMesh(core_axis_name="c", subcore_axis_name="s"),
    compiler_params=pltpu.CompilerParams(needs_layout_passes=False),
    scratch_types=[
        pltpu.VMEM((EMB, VBLK + 1), jnp.float32),
        pltpu.VMEM((EMB, VBLK + 1), jnp.float32),
        pltpu.VMEM((_TROWS, VBLK), jnp.float32),
        pltpu.VMEM((_TROWS, VBLK), jnp.float32),
        pltpu.SemaphoreType.DMA,
        pltpu.SemaphoreType.DMA,
        pltpu.SemaphoreType.DMA,
        pltpu.SemaphoreType.DMA,
    ],
)
def _transpose_table(tab_t_hbm, tail_hbm, out_hbm,
                     src0, src1, dst0, dst1, in0, in1, out0, out1):
    wid = lax.axis_index("s") * NC + lax.axis_index("c")
    iota = lax.iota(jnp.int32, LANES)
    srcs, dsts = (src0, src1), (dst0, dst1)
    in_sems, out_sems = (in0, in1), (out0, out1)
    e_vecs = [LANES * jj + iota for jj in range(NCOL)]

    def start_in(c, p):
        pltpu.async_copy(
            tab_t_hbm.at[:, pl.ds(c * VBLK, VBLK)],
            srcs[p].at[:, pl.ds(0, VBLK)],
            in_sems[p],
        )

    def transpose_block(src, dst):
        @plsc.parallel_loop(0, _TROWS // _TUNROLL, 1, unroll=2)
        def _(it):
            rows = []
            for rr in range(_TUNROLL):
                u0v = jnp.full((LANES,), 2 * (it * _TUNROLL + rr), jnp.int32)
                u1v = u0v + 1
                rows.append([
                    plsc.load_gather(
                        src, [e_vecs[j % NCOL], u0v if j < NCOL else u1v]
                    )
                    for j in range(2 * NCOL)
                ])
            for rr in range(_TUNROLL):
                r = it * _TUNROLL + rr
                for j in range(2 * NCOL):
                    dst[r, pl.ds(LANES * j, LANES)] = rows[rr][j]

    start_in(wid, 0)
    start_in(NW + wid, 1)

    def body(m, carry):
        for p in range(2):
            c = (2 * m + p) * NW + wid

            @pl.when(c < NFULL)
            def _():
                pltpu.make_async_copy(
                    tab_t_hbm.at[:, pl.ds(c * VBLK, VBLK)],
                    srcs[p].at[:, pl.ds(0, VBLK)],
                    in_sems[p],
                ).wait()

                @pl.when(m >= 1)
                def _():
                    pltpu.make_async_copy(
                        dsts[p], out_hbm.at[pl.ds(0, _TROWS), :], out_sems[p]
                    ).wait()

                transpose_block(srcs[p], dsts[p])
                pltpu.async_copy(
                    dsts[p],
                    out_hbm.at[pl.ds(c * _TROWS, _TROWS), :],
                    out_sems[p],
                )
                cn = c + 2 * NW

                @pl.when(cn < NFULL)
                def _():
                    start_in(cn, p)

        return carry

    lax.fori_loop(0, (BLK_PER_W + 1) // 2, body, 0)

    for p in range(2):
        @pl.when(p * NW + wid < NFULL)
        def _():
            pltpu.make_async_copy(
                dsts[p], out_hbm.at[pl.ds(0, _TROWS), :], out_sems[p]
            ).wait()

    @pl.when(wid == 0)
    def _():
        pltpu.sync_copy(tail_hbm, dst0.at[pl.ds(0, TAIL // 2), :])
        pltpu.sync_copy(
            dst0.at[pl.ds(0, TAIL // 2), :],
            out_hbm.at[pl.ds(NFULL * (VBLK // 2), TAIL // 2), :],
        )


def _reduce_chunk(buf):
    def body(it, accs):
        s0 = it * _UNROLL
        for u in range(_UNROLL):
            accs = tuple(
                jnp.maximum(a, buf[s0 + u, pl.ds(LANES * j, LANES)])
                for j, a in enumerate(accs)
            )
        return accs

    init = tuple(jnp.full((LANES,), _NEG, jnp.float32) for _ in range(NCOL))
    return lax.fori_loop(0, CHUNK // _UNROLL, body, init)


@functools.partial(
    pl.kernel,
    out_type=jax.ShapeDtypeStruct((BATCH, EMB), jnp.float32),
    mesh=plsc.VectorSubcoreMesh(core_axis_name="c", subcore_axis_name="s"),
    compiler_params=pltpu.CompilerParams(use_tc_tiling_on_sc=False),
    scratch_types=[
        pltpu.VMEM((ROWS_PER_W, CHUNK), jnp.int32),
        pltpu.VMEM((CHUNK, EMB), jnp.float32),
        pltpu.VMEM((CHUNK, EMB), jnp.float32),
        pltpu.VMEM((B_PER_W, EMB), jnp.float32),
        pltpu.SemaphoreType.DMA,
        pltpu.SemaphoreType.DMA,
    ],
)
def _bow_encode(idx_hbm, table_hbm, out_hbm, idx_v, buf0, buf1, out_v, sem0, sem1):
    wid = lax.axis_index("s") * NC + lax.axis_index("c")
    base = wid * ROWS_PER_W

    pltpu.sync_copy(idx_hbm.at[pl.ds(base, ROWS_PER_W), :], idx_v)

    pltpu.async_copy(table_hbm.at[idx_v.at[0]], buf0, sem0)
    pltpu.async_copy(table_hbm.at[idx_v.at[1]], buf1, sem1)

    def gbody(g, carry):
        r0 = 2 * g

        pltpu.make_async_copy(table_hbm.at[idx_v.at[r0]], buf0, sem0).wait()
        acc0 = _reduce_chunk(buf0)

        @pl.when(g < B_PER_W - 1)
        def _():
            pltpu.async_copy(table_hbm.at[idx_v.at[r0 + 2]], buf0, sem0)

        pltpu.make_async_copy(table_hbm.at[idx_v.at[r0 + 1]], buf1, sem1).wait()
        acc1 = _reduce_chunk(buf1)

        @pl.when(g < B_PER_W - 1)
        def _():
            pltpu.async_copy(table_hbm.at[idx_v.at[r0 + 3]], buf1, sem1)

        for j in range(NCOL):
            out_v[g, pl.ds(LANES * j, LANES)] = jnp.maximum(acc0[j], acc1[j])
        return carry

    lax.fori_loop(0, B_PER_W, gbody, 0)

    pltpu.sync_copy(out_v, out_hbm.at[pl.ds(wid * B_PER_W, B_PER_W), :])


@jax.jit
def kernel(input, emb_weight):
    idx = input.astype(jnp.int32).reshape(BATCH * CHUNKS_PER_B, CHUNK)
    tail = emb_weight[NFULL * VBLK :, :].reshape(TAIL // 2, 2 * EMB)
    ctable = _transpose_table(emb_weight.T, tail)
    return _bow_encode(idx, ctable.reshape(VOCAB, EMB))


# speedup vs baseline: 2.1727x; 1.5126x over previous
import functools

import jax
import jax.numpy as jnp
from jax import lax
from jax.experimental import pallas as pl
from jax.experimental.pallas import tpu as pltpu
from jax.experimental.pallas import tpu_sc as plsc

BATCH = 4096
SEQ = 200
EMB = 64
LANES = 16
NCOL = EMB // LANES

NC = 2
NS = 16
NW = NC * NS
B_PER_W = BATCH // NW
CHUNKS_PER_B = 2
CHUNK = SEQ // CHUNKS_PER_B
ROWS_PER_W = B_PER_W * CHUNKS_PER_B

_NEG = float(jnp.finfo(jnp.float32).min)
_UNROLL = 20


def _reduce_chunk(buf):
    def body(it, accs):
        s0 = it * _UNROLL
        for u in range(_UNROLL):
            accs = tuple(
                jnp.maximum(a, buf[s0 + u, pl.ds(LANES * j, LANES)])
                for j, a in enumerate(accs)
            )
        return accs

    init = tuple(jnp.full((LANES,), _NEG, jnp.float32) for _ in range(NCOL))
    return lax.fori_loop(0, CHUNK // _UNROLL, body, init)


@functools.partial(
    pl.kernel,
    out_type=jax.ShapeDtypeStruct((BATCH, EMB), jnp.float32),
    mesh=plsc.VectorSubcoreMesh(core_axis_name="c", subcore_axis_name="s"),
    compiler_params=pltpu.CompilerParams(use_tc_tiling_on_sc=False),
    scratch_types=[
        pltpu.VMEM((ROWS_PER_W, CHUNK), jnp.int32),
        pltpu.VMEM((CHUNK, EMB), jnp.float32),
        pltpu.VMEM((CHUNK, EMB), jnp.float32),
        pltpu.VMEM((B_PER_W, EMB), jnp.float32),
        pltpu.SemaphoreType.DMA,
        pltpu.SemaphoreType.DMA,
    ],
)
def _bow_encode(idx_hbm, table_hbm, out_hbm, idx_v, buf0, buf1, out_v, sem0, sem1):
    wid = lax.axis_index("s") * NC + lax.axis_index("c")
    base = wid * ROWS_PER_W

    pltpu.sync_copy(idx_hbm.at[pl.ds(base, ROWS_PER_W), :], idx_v)

    pltpu.async_copy(table_hbm.at[idx_v.at[0]], buf0, sem0)
    pltpu.async_copy(table_hbm.at[idx_v.at[1]], buf1, sem1)

    def gbody(g, carry):
        r0 = 2 * g

        pltpu.make_async_copy(table_hbm.at[idx_v.at[r0]], buf0, sem0).wait()
        acc0 = _reduce_chunk(buf0)

        @pl.when(g < B_PER_W - 1)
        def _():
            pltpu.async_copy(table_hbm.at[idx_v.at[r0 + 2]], buf0, sem0)

        pltpu.make_async_copy(table_hbm.at[idx_v.at[r0 + 1]], buf1, sem1).wait()
        acc1 = _reduce_chunk(buf1)

        @pl.when(g < B_PER_W - 1)
        def _():
            pltpu.async_copy(table_hbm.at[idx_v.at[r0 + 3]], buf1, sem1)

        for j in range(NCOL):
            out_v[g, pl.ds(LANES * j, LANES)] = jnp.maximum(acc0[j], acc1[j])
        return carry

    lax.fori_loop(0, B_PER_W, gbody, 0)

    pltpu.sync_copy(out_v, out_hbm.at[pl.ds(wid * B_PER_W, B_PER_W), :])


@jax.jit
def kernel(input, emb_weight):
    idx = input.astype(jnp.int32).reshape(BATCH * CHUNKS_PER_B, CHUNK)
    return _bow_encode(idx, emb_weight)


# speedup vs baseline: 2.3818x; 1.0963x over previous
import functools

import jax
import jax.numpy as jnp
from jax import lax
from jax.experimental import pallas as pl
from jax.experimental.pallas import tpu as pltpu
from jax.experimental.pallas import tpu_sc as plsc

BATCH = 4096
SEQ = 200
EMB = 64
LANES = 16
NCOL = EMB // LANES

NC = 2
NS = 16
NW = NC * NS
B_PER_W = BATCH // NW
CHUNKS_PER_B = 2
CHUNK = SEQ // CHUNKS_PER_B
ROWS_PER_W = B_PER_W * CHUNKS_PER_B

_NEG = float(jnp.finfo(jnp.float32).min)
_UNROLL = 20


def _reduce_chunk(buf):
    def body(it, accs):
        s0 = it * _UNROLL
        for u in range(_UNROLL):
            accs = tuple(
                jnp.maximum(a, buf[s0 + u, pl.ds(LANES * j, LANES)])
                for j, a in enumerate(accs)
            )
        return accs

    init = tuple(jnp.full((LANES,), _NEG, jnp.float32) for _ in range(NCOL))
    return lax.fori_loop(0, CHUNK // _UNROLL, body, init)


@functools.partial(
    pl.kernel,
    out_type=jax.ShapeDtypeStruct((BATCH, EMB), jnp.float32),
    mesh=plsc.VectorSubcoreMesh(core_axis_name="c", subcore_axis_name="s"),
    compiler_params=pltpu.CompilerParams(use_tc_tiling_on_sc=False),
    scratch_types=[
        pltpu.VMEM((ROWS_PER_W, CHUNK), jnp.int32),
        pltpu.VMEM((CHUNK, EMB), jnp.float32),
        pltpu.VMEM((CHUNK, EMB), jnp.float32),
        pltpu.VMEM((CHUNK, EMB), jnp.float32),
        pltpu.VMEM((CHUNK, EMB), jnp.float32),
        pltpu.VMEM((B_PER_W, EMB), jnp.float32),
        pltpu.SemaphoreType.DMA,
        pltpu.SemaphoreType.DMA,
        pltpu.SemaphoreType.DMA,
        pltpu.SemaphoreType.DMA,
    ],
)
def _bow_encode(idx_hbm, table_hbm, out_hbm, idx_v,
                buf0, buf1, buf2, buf3, out_v, sem0, sem1, sem2, sem3):
    wid = lax.axis_index("s") * NC + lax.axis_index("c")
    base = wid * ROWS_PER_W
    bufs = (buf0, buf1, buf2, buf3)
    sems = (sem0, sem1, sem2, sem3)
    nbuf = len(bufs)

    pltpu.sync_copy(idx_hbm.at[pl.ds(base, ROWS_PER_W), :], idx_v)

    for q in range(nbuf):
        pltpu.async_copy(table_hbm.at[idx_v.at[q]], bufs[q], sems[q])

    def gbody(h, carry):
        accs = []
        for q in range(nbuf):
            r = nbuf * h + q
            pltpu.make_async_copy(table_hbm.at[idx_v.at[r]], bufs[q], sems[q]).wait()
            accs.append(_reduce_chunk(bufs[q]))

            @pl.when(h < B_PER_W // 2 - 1)
            def _():
                pltpu.async_copy(table_hbm.at[idx_v.at[r + nbuf]], bufs[q], sems[q])

        for j in range(NCOL):
            out_v[2 * h, pl.ds(LANES * j, LANES)] = jnp.maximum(accs[0][j], accs[1][j])
            out_v[2 * h + 1, pl.ds(LANES * j, LANES)] = jnp.maximum(accs[2][j], accs[3][j])
        return carry

    lax.fori_loop(0, B_PER_W // 2, gbody, 0)

    pltpu.sync_copy(out_v, out_hbm.at[pl.ds(wid * B_PER_W, B_PER_W), :])


@jax.jit
def kernel(input, emb_weight):
    idx = input.astype(jnp.int32).reshape(BATCH * CHUNKS_PER_B, CHUNK)
    return _bow_encode(idx, emb_weight)
